# baseline (speedup 1.0000x reference)
import functools

import jax
import jax.numpy as jnp
from jax import lax
from jax.experimental import pallas as pl
from jax.experimental.pallas import tpu as pltpu
from jax.experimental.pallas import tpu_sc as plsc

N_BANDS = 6
QUANTILE = 0.02
LANES = 16
SUBCOLS = 256
NSUB = 4


def _ce(a, b):
    return jnp.minimum(a, b), jnp.maximum(a, b)


def _ins_top(t1, t2, t3, v):
    a = jnp.maximum(t1, v)
    b = jnp.minimum(t1, v)
    c = jnp.maximum(t2, b)
    d = jnp.minimum(t2, b)
    e = jnp.maximum(t3, d)
    return a, c, e


def _ins_bot(n1, n2, n3, v):
    a = jnp.minimum(n1, v)
    b = jnp.maximum(n1, v)
    c = jnp.minimum(n2, b)
    d = jnp.maximum(n2, b)
    e = jnp.minimum(n3, d)
    return a, c, e


def _quartet(st, x0, x1, x2, x3):
    t1, t2, t3, n1, n2, n3 = st
    s0, s1 = _ce(x0, x1)
    s2, s3 = _ce(x2, x3)
    s0, s2 = _ce(s0, s2)
    s1, s3 = _ce(s1, s3)
    s1, s2 = _ce(s1, s2)
    t1, t2, t3 = _ins_top(t1, t2, t3, s3)
    c2 = jnp.maximum(t2, s2)
    d2 = jnp.minimum(t2, s2)
    t3 = jnp.maximum(t3, d2)
    t2 = c2
    t3 = jnp.maximum(t3, s1)
    n1, n2, n3 = _ins_bot(n1, n2, n3, s0)
    c2 = jnp.minimum(n2, s1)
    d2 = jnp.maximum(n2, s1)
    n3 = jnp.minimum(n3, d2)
    n2 = c2
    n3 = jnp.minimum(n3, s2)
    return t1, t2, t3, n1, n2, n3


@functools.lru_cache(maxsize=None)
def _build(B, F, T):
    band_size = F // N_BANDS
    starts = [i * band_size for i in range(N_BANDS)]
    ends = [(i + 1) * band_size for i in range(N_BANDS - 1)] + [F]
    for s, e in zip(starts, ends):
        if max(1, int((e - s) * QUANTILE)) != 3:
            raise NotImplementedError("selection network is specialized for k == 3")
    al = [(s // 8) * 8 for s in starts]
    ro = [s - a for s, a in zip(starts, al)]
    sz = [176] * N_BANDS
    max_sz = max(sz)
    main_rows = [e - s for s, e in zip(starts, ends)]
    main_rows[N_BANDS - 1] -= 1

    info = plsc.get_sparse_core_info()
    n_workers = info.num_cores * info.num_subcores
    slab = T * B // n_workers
    out_len = B * N_BANDS * T

    mesh = plsc.VectorSubcoreMesh(core_axis_name="c", subcore_axis_name="s")

    @functools.partial(
        pl.kernel,
        mesh=mesh,
        out_type=[
            jax.ShapeDtypeStruct((out_len,), jnp.float32),
            jax.ShapeDtypeStruct((out_len,), jnp.float32),
        ],
        scratch_types=[
            pltpu.VMEM((max_sz, SUBCOLS), jnp.float32),
            pltpu.VMEM((max_sz, SUBCOLS), jnp.float32),
            pltpu.VMEM((1, SUBCOLS), jnp.float32),
            pltpu.VMEM((slab,), jnp.float32),
            pltpu.VMEM((slab,), jnp.float32),
            pltpu.VMEM((slab,), jnp.float32),
            pltpu.VMEM((slab,), jnp.float32),
            pltpu.SemaphoreType.DMA,
            pltpu.SemaphoreType.DMA,
            pltpu.SemaphoreType.DMA,
            pltpu.SemaphoreType.DMA,
        ],
    )
    def sc_kernel(x_hbm, peaks_hbm, valleys_hbm,
                  buf0, buf1, lastbuf, pk0, pk1, vl0, vl1,
                  insem0, insem1, outsem0, outsem1):
        wid = lax.axis_index("s") * info.num_cores + lax.axis_index("c")
        slabs_per_b = T // slab
        b = wid // slabs_per_b
        col0 = (wid % slabs_per_b) * slab

        bufs = (buf0, buf1)
        insems = (insem0, insem1)
        pks = (pk0, pk1)
        vls = (vl0, vl1)
        outsems = (outsem0, outsem1)

        def start_in(band, sub, slot):
            return pltpu.async_copy(
                x_hbm.at[b, pl.ds(al[band], sz[band]),
                         pl.ds(col0 + sub * SUBCOLS, SUBCOLS)],
                bufs[slot].at[pl.ds(0, sz[band])], insems[slot])

        seq = [(band, sub) for band in range(N_BANDS) for sub in range(NSUB)]
        pending_in = {0: start_in(0, 0, 0)}
        pending_out = {}

        for k, (band, sub) in enumerate(seq):
            slot = k % 2
            nrows = main_rows[band]
            r_lo = ro[band]
            has_last = band == N_BANDS - 1

            pending_in.pop(slot).wait()
            if k + 1 < len(seq):
                nband, nsub = seq[k + 1]
                pending_in[1 - slot] = start_in(nband, nsub, 1 - slot)

            if sub == 0:
                bslot = band % 2
                if bslot in pending_out:
                    for h in pending_out.pop(bslot):
                        h.wait()

            buf = bufs[slot]
            bslot = band % 2
            pkb, vlb = pks[bslot], vls[bslot]

            if has_last:
                pltpu.sync_copy(
                    x_hbm.at[b, pl.ds(F - 1, 1),
                             pl.ds(col0 + sub * SUBCOLS, SUBCOLS)],
                    lastbuf)

            UNROLL = 16
            nblk = nrows // UNROLL
            tail = nrows - nblk * UNROLL
            tail_q = tail // 4
            tail_s = tail - tail_q * 4

            def group(g, carry, buf=buf, sub=sub, r_lo=r_lo, nblk=nblk,
                      tail_q=tail_q, tail_s=tail_s, has_last=has_last):
                c0 = g * LANES
                neg = jnp.full((LANES,), -jnp.inf, jnp.float32)
                pos = jnp.full((LANES,), jnp.inf, jnp.float32)

                def block16(ci, st, buf=buf, c0=c0, r_lo=r_lo):
                    r0 = r_lo + ci * UNROLL
                    for q in range(UNROLL // 4):
                        st = _quartet(
                            st,
                            buf[r0 + 4 * q, pl.ds(c0, LANES)],
                            buf[r0 + 4 * q + 1, pl.ds(c0, LANES)],
                            buf[r0 + 4 * q + 2, pl.ds(c0, LANES)],
                            buf[r0 + 4 * q + 3, pl.ds(c0, LANES)],
                        )
                    return st

                st = lax.fori_loop(0, nblk, block16, (neg, neg, neg, pos, pos, pos))
                rbase = r_lo + nblk * UNROLL
                for q in range(tail_q):
                    st = _quartet(
                        st,
                        buf[rbase + 4 * q, pl.ds(c0, LANES)],
                        buf[rbase + 4 * q + 1, pl.ds(c0, LANES)],
                        buf[rbase + 4 * q + 2, pl.ds(c0, LANES)],
                        buf[rbase + 4 * q + 3, pl.ds(c0, LANES)],
                    )
                t1, t2, t3, n1, n2, n3 = st
                for j in range(tail_s):
                    v = buf[rbase + tail_q * 4 + j, pl.ds(c0, LANES)]
                    t1, t2, t3 = _ins_top(t1, t2, t3, v)
                    n1, n2, n3 = _ins_bot(n1, n2, n3, v)
                if has_last:
                    v = lastbuf[0, pl.ds(c0, LANES)]
                    t1, t2, t3 = _ins_top(t1, t2, t3, v)
                    n1, n2, n3 = _ins_bot(n1, n2, n3, v)
                o0 = sub * SUBCOLS + c0
                pkb[pl.ds(o0, LANES)] = (t1 + t2 + t3) / 3.0
                vlb[pl.ds(o0, LANES)] = (n1 + n2 + n3) / 3.0
                return carry

            lax.fori_loop(0, SUBCOLS // LANES, group, 0)

            if sub == NSUB - 1:
                obase = (b * N_BANDS + band) * T + col0
                pending_out[bslot] = (
                    pltpu.async_copy(pkb, peaks_hbm.at[pl.ds(obase, slab)],
                                     outsems[bslot]),
                    pltpu.async_copy(vlb, valleys_hbm.at[pl.ds(obase, slab)],
                                     outsems[bslot]),
                )

        for hs in pending_out.values():
            for h in hs:
                h.wait()

    return sc_kernel


def kernel(power_spec):
    B, F, T = power_spec.shape
    Th = T // 2
    sc_kernel = _build(B, F, Th)
    pa, va = sc_kernel(power_spec[:, :, :Th])
    pb, vb = sc_kernel(power_spec[:, :, Th:])
    peaks = jnp.concatenate(
        [pa.reshape(B, N_BANDS, Th), pb.reshape(B, N_BANDS, Th)], axis=2)
    valleys = jnp.concatenate(
        [va.reshape(B, N_BANDS, Th), vb.reshape(B, N_BANDS, Th)], axis=2)
    return (peaks, valleys)


# speedup vs baseline: 1.8505x; 1.8505x over previous
import functools

import jax
import jax.numpy as jnp
from jax import lax
from jax.experimental import pallas as pl
from jax.experimental.pallas import tpu as pltpu
from jax.experimental.pallas import tpu_sc as plsc

N_BANDS = 6
QUANTILE = 0.02
LANES = 16
SUBCOLS = 256
NSUB = 4


def _ce(a, b):
    return jnp.minimum(a, b), jnp.maximum(a, b)


def _ins_top(t1, t2, t3, v):
    a = jnp.maximum(t1, v)
    b = jnp.minimum(t1, v)
    c = jnp.maximum(t2, b)
    d = jnp.minimum(t2, b)
    e = jnp.maximum(t3, d)
    return a, c, e


def _ins_bot(n1, n2, n3, v):
    a = jnp.minimum(n1, v)
    b = jnp.maximum(n1, v)
    c = jnp.minimum(n2, b)
    d = jnp.maximum(n2, b)
    e = jnp.minimum(n3, d)
    return a, c, e


def _quartet(st, x0, x1, x2, x3):
    t1, t2, t3, n1, n2, n3 = st
    s0, s1 = _ce(x0, x1)
    s2, s3 = _ce(x2, x3)
    s0, s2 = _ce(s0, s2)
    s1, s3 = _ce(s1, s3)
    s1, s2 = _ce(s1, s2)
    t1, t2, t3 = _ins_top(t1, t2, t3, s3)
    c2 = jnp.maximum(t2, s2)
    d2 = jnp.minimum(t2, s2)
    t3 = jnp.maximum(t3, d2)
    t2 = c2
    t3 = jnp.maximum(t3, s1)
    n1, n2, n3 = _ins_bot(n1, n2, n3, s0)
    c2 = jnp.minimum(n2, s1)
    d2 = jnp.maximum(n2, s1)
    n3 = jnp.minimum(n3, d2)
    n2 = c2
    n3 = jnp.minimum(n3, s2)
    return t1, t2, t3, n1, n2, n3


@functools.lru_cache(maxsize=None)
def _build(B, F, T):
    band_size = F // N_BANDS
    starts = [i * band_size for i in range(N_BANDS)]
    ends = [(i + 1) * band_size for i in range(N_BANDS - 1)] + [F]
    for s, e in zip(starts, ends):
        if max(1, int((e - s) * QUANTILE)) != 3:
            raise NotImplementedError("selection network is specialized for k == 3")
    al = [(s // 8) * 8 for s in starts]
    ro = [s - a for s, a in zip(starts, al)]
    sz = [176] * N_BANDS
    max_sz = max(sz)
    main_rows = [e - s for s, e in zip(starts, ends)]
    main_rows[N_BANDS - 1] -= 1

    info = plsc.get_sparse_core_info()
    n_workers = info.num_cores * info.num_subcores
    slab = T * B // n_workers
    out_len = B * N_BANDS * T

    mesh = plsc.VectorSubcoreMesh(core_axis_name="c", subcore_axis_name="s")

    @functools.partial(
        pl.kernel,
        mesh=mesh,
        out_type=[
            jax.ShapeDtypeStruct((out_len,), jnp.float32),
            jax.ShapeDtypeStruct((out_len,), jnp.float32),
        ],
        scratch_types=[
            pltpu.VMEM((max_sz, SUBCOLS), jnp.float32),
            pltpu.VMEM((max_sz, SUBCOLS), jnp.float32),
            pltpu.VMEM((1, SUBCOLS), jnp.float32),
            pltpu.VMEM((slab,), jnp.float32),
            pltpu.VMEM((slab,), jnp.float32),
            pltpu.VMEM((slab,), jnp.float32),
            pltpu.VMEM((slab,), jnp.float32),
            pltpu.SemaphoreType.DMA,
            pltpu.SemaphoreType.DMA,
            pltpu.SemaphoreType.DMA,
            pltpu.SemaphoreType.DMA,
        ],
    )
    def sc_kernel(x_hbm, peaks_hbm, valleys_hbm,
                  buf0, buf1, lastbuf, pk0, pk1, vl0, vl1,
                  insem0, insem1, outsem0, outsem1):
        wid = lax.axis_index("s") * info.num_cores + lax.axis_index("c")
        slabs_per_b = T // slab
        b = wid // slabs_per_b
        col0 = (wid % slabs_per_b) * slab

        bufs = (buf0, buf1)
        insems = (insem0, insem1)
        pks = (pk0, pk1)
        vls = (vl0, vl1)
        outsems = (outsem0, outsem1)

        def start_in(band, sub, slot):
            return pltpu.async_copy(
                x_hbm.at[b, pl.ds(al[band], sz[band]),
                         pl.ds(col0 + sub * SUBCOLS, SUBCOLS)],
                bufs[slot].at[pl.ds(0, sz[band])], insems[slot])

        seq = [(band, sub) for band in range(N_BANDS) for sub in range(NSUB)]
        pending_in = {0: start_in(0, 0, 0)}
        pending_out = {}

        for k, (band, sub) in enumerate(seq):
            slot = k % 2
            nrows = main_rows[band]
            r_lo = ro[band]
            has_last = band == N_BANDS - 1

            pending_in.pop(slot).wait()
            if k + 1 < len(seq):
                nband, nsub = seq[k + 1]
                pending_in[1 - slot] = start_in(nband, nsub, 1 - slot)

            if sub == 0:
                bslot = band % 2
                if bslot in pending_out:
                    for h in pending_out.pop(bslot):
                        h.wait()

            buf = bufs[slot]
            bslot = band % 2
            pkb, vlb = pks[bslot], vls[bslot]

            if has_last:
                pltpu.sync_copy(
                    x_hbm.at[b, pl.ds(F - 1, 1),
                             pl.ds(col0 + sub * SUBCOLS, SUBCOLS)],
                    lastbuf)

            UNROLL = 16
            nblk = nrows // UNROLL
            tail = nrows - nblk * UNROLL
            tail_q = tail // 4
            tail_s = tail - tail_q * 4

            def group(g, carry, buf=buf, sub=sub, r_lo=r_lo, nblk=nblk,
                      tail_q=tail_q, tail_s=tail_s, has_last=has_last):
                c0 = g * LANES
                neg = jnp.full((LANES,), -jnp.inf, jnp.float32)
                pos = jnp.full((LANES,), jnp.inf, jnp.float32)

                def block16(ci, st, buf=buf, c0=c0, r_lo=r_lo):
                    r0 = r_lo + ci * UNROLL
                    for q in range(UNROLL // 4):
                        st = _quartet(
                            st,
                            buf[r0 + 4 * q, pl.ds(c0, LANES)],
                            buf[r0 + 4 * q + 1, pl.ds(c0, LANES)],
                            buf[r0 + 4 * q + 2, pl.ds(c0, LANES)],
                            buf[r0 + 4 * q + 3, pl.ds(c0, LANES)],
                        )
                    return st

                st = lax.fori_loop(0, nblk, block16, (neg, neg, neg, pos, pos, pos))
                rbase = r_lo + nblk * UNROLL
                for q in range(tail_q):
                    st = _quartet(
                        st,
                        buf[rbase + 4 * q, pl.ds(c0, LANES)],
                        buf[rbase + 4 * q + 1, pl.ds(c0, LANES)],
                        buf[rbase + 4 * q + 2, pl.ds(c0, LANES)],
                        buf[rbase + 4 * q + 3, pl.ds(c0, LANES)],
                    )
                t1, t2, t3, n1, n2, n3 = st
                for j in range(tail_s):
                    v = buf[rbase + tail_q * 4 + j, pl.ds(c0, LANES)]
                    t1, t2, t3 = _ins_top(t1, t2, t3, v)
                    n1, n2, n3 = _ins_bot(n1, n2, n3, v)
                if has_last:
                    v = lastbuf[0, pl.ds(c0, LANES)]
                    t1, t2, t3 = _ins_top(t1, t2, t3, v)
                    n1, n2, n3 = _ins_bot(n1, n2, n3, v)
                o0 = sub * SUBCOLS + c0
                pkb[pl.ds(o0, LANES)] = (t1 + t2 + t3) / 3.0
                vlb[pl.ds(o0, LANES)] = (n1 + n2 + n3) / 3.0
                return carry

            lax.fori_loop(0, SUBCOLS // LANES, group, 0)

            if sub == NSUB - 1:
                obase = (b * N_BANDS + band) * T + col0
                pending_out[bslot] = (
                    pltpu.async_copy(pkb, peaks_hbm.at[pl.ds(obase, slab)],
                                     outsems[bslot]),
                    pltpu.async_copy(vlb, valleys_hbm.at[pl.ds(obase, slab)],
                                     outsems[bslot]),
                )

        for hs in pending_out.values():
            for h in hs:
                h.wait()

    return sc_kernel


def kernel(power_spec):
    B, F, T = power_spec.shape
    sc_kernel = _build(B, F, T)
    peaks, valleys = sc_kernel(power_spec)
    return (
        peaks.reshape(B, N_BANDS, T),
        valleys.reshape(B, N_BANDS, T),
    )
